# Initial kernel scaffold; baseline (speedup 1.0000x reference)
import jax
import jax.numpy as jnp
from jax.experimental import pallas as pl


def kernel(x, w1, b1, w2, b2, w3, b3):
    raise NotImplementedError("write your pallas kernel here")


# speedup vs baseline: 1.1379x; 1.1379x over previous
import jax
import jax.numpy as jnp
from jax.experimental import pallas as pl
from jax.experimental.pallas import tpu as pltpu

IN_FEATURES = 3
H1 = 64
H2 = 32
NUM_CLASSES = 5

PACK = 8
ROW_TILE = 2000


def _round_up(n, m):
    return m * pl.cdiv(n, m)


def _block_diag(w, p):
    k, n = w.shape
    eye = jnp.eye(p, dtype=w.dtype)
    return (eye[:, None, :, None] * w[None, :, None, :]).reshape(p * k, p * n)


def _packed_mlp_kernel(x_ref, w1_ref, b1_ref, w2_ref, b2_ref, w3_ref, b3_ref,
                       o_ref):
    x = x_ref[...]
    h1 = jnp.dot(x, w1_ref[...], preferred_element_type=jnp.float32)
    h1 = jnp.maximum(h1 + b1_ref[...], 0.0)
    h2 = jnp.dot(h1, w2_ref[...], preferred_element_type=jnp.float32)
    h2 = jnp.maximum(h2 + b2_ref[...], 0.0)
    out = jnp.dot(h2, w3_ref[...], preferred_element_type=jnp.float32)
    o_ref[...] = (out + b3_ref[...]).astype(o_ref.dtype)


@jax.jit
def kernel(x, w1, b1, w2, b2, w3, b3):
    B = x.shape[0]
    p = PACK

    w1t = _block_diag(w1[:IN_FEATURES], p)
    w2t = _block_diag(w2, p)
    w3t = _block_diag(w3, p)
    b1t = jnp.tile(b1, (1, p))
    b2t = jnp.tile(b2, (1, p))
    b3t = jnp.tile(b3, (1, p))

    padded_b = _round_up(B, p)
    if padded_b != B:
        x = jnp.pad(x, ((0, padded_b - B), (0, 0)))
    rows = padded_b // p
    xp = x.reshape(rows, p * IN_FEATURES)

    rt = min(ROW_TILE, _round_up(rows, 8))
    grid = (pl.cdiv(rows, rt),)

    def batch_map(i):
        return (i, 0)

    def const_map(i):
        return (0, 0)

    out = pl.pallas_call(
        _packed_mlp_kernel,
        out_shape=jax.ShapeDtypeStruct((rows, p * NUM_CLASSES), jnp.float32),
        grid=grid,
        in_specs=[
            pl.BlockSpec((rt, p * IN_FEATURES), batch_map),
            pl.BlockSpec((p * IN_FEATURES, p * H1), const_map),
            pl.BlockSpec((1, p * H1), const_map),
            pl.BlockSpec((p * H1, p * H2), const_map),
            pl.BlockSpec((1, p * H2), const_map),
            pl.BlockSpec((p * H2, p * NUM_CLASSES), const_map),
            pl.BlockSpec((1, p * NUM_CLASSES), const_map),
        ],
        out_specs=pl.BlockSpec((rt, p * NUM_CLASSES), batch_map),
        compiler_params=pltpu.CompilerParams(
            dimension_semantics=("parallel",)),
    )(xp, w1t, b1t, w2t, b2t, w3t, b3t)

    out = out.reshape(padded_b, NUM_CLASSES)
    return out if padded_b == B else out[:B]


# speedup vs baseline: 4.5789x; 4.0239x over previous
import jax
import jax.numpy as jnp
from jax.experimental import pallas as pl
from jax.experimental.pallas import tpu as pltpu

IN_FEATURES = 3
H1 = 64
H2 = 32
NUM_CLASSES = 5

BATCH_TILE = 4096


def _round_up(n, m):
    return m * pl.cdiv(n, m)


def _mlp_kernel(x_ref, w1_ref, b1_ref, w2_ref, b2_ref, w3_ref, b3_ref, o_ref):
    x = x_ref[...]
    h1 = jnp.dot(x, w1_ref[...], preferred_element_type=jnp.float32)
    h1 = jnp.maximum(h1 + b1_ref[...], 0.0)
    h2 = jnp.dot(h1, w2_ref[...], preferred_element_type=jnp.float32)
    h2 = jnp.maximum(h2 + b2_ref[...], 0.0)
    out = jnp.dot(h2, w3_ref[...], preferred_element_type=jnp.float32)
    o_ref[...] = (out + b3_ref[...]).astype(o_ref.dtype)


@jax.jit
def kernel(x, w1, b1, w2, b2, w3, b3):
    B = x.shape[0]

    w1c = w1[:IN_FEATURES]

    bt = min(BATCH_TILE, _round_up(B, 8))
    grid = (pl.cdiv(B, bt),)

    def batch_map(i):
        return (i, 0)

    def const_map(i):
        return (0, 0)

    return pl.pallas_call(
        _mlp_kernel,
        out_shape=jax.ShapeDtypeStruct((B, NUM_CLASSES), jnp.float32),
        grid=grid,
        in_specs=[
            pl.BlockSpec((bt, IN_FEATURES), batch_map),
            pl.BlockSpec((IN_FEATURES, H1), const_map),
            pl.BlockSpec((1, H1), const_map),
            pl.BlockSpec((H1, H2), const_map),
            pl.BlockSpec((1, H2), const_map),
            pl.BlockSpec((H2, NUM_CLASSES), const_map),
            pl.BlockSpec((1, NUM_CLASSES), const_map),
        ],
        out_specs=pl.BlockSpec((bt, NUM_CLASSES), batch_map),
        compiler_params=pltpu.CompilerParams(
            dimension_semantics=("parallel",)),
    )(x, w1c, b1, w2, b2, w3, b3)
